# baseline (speedup 1.0000x reference)
import functools

import jax
import jax.numpy as jnp
from jax import lax
from jax.experimental import pallas as pl
from jax.experimental.pallas import tpu as pltpu
from jax.experimental.pallas import tpu_sc as plsc

_NC = 2
_NS = 16
_LANES = 16
_PAD = 64


def _sc_tail_sum(hidden2, mask_flat, xs, B, L, D):
    Dc = D // _NC
    nchunk = Dc // _LANES
    mesh = plsc.VectorSubcoreMesh(core_axis_name="c", subcore_axis_name="s")

    @functools.partial(
        pl.kernel,
        out_type=jax.ShapeDtypeStruct((B * D,), jnp.float32),
        mesh=mesh,
        compiler_params=pltpu.CompilerParams(needs_layout_passes=False),
        scratch_types=[
            pltpu.VMEM((L,), jnp.int32),
            pltpu.VMEM((_LANES,), jnp.int32),
            pltpu.VMEM((_PAD,), jnp.int32),
            pltpu.VMEM((_PAD, Dc), jnp.float32),
            pltpu.VMEM((Dc,), jnp.float32),
            pltpu.SemaphoreType.DMA,
        ],
    )
    def k(hidden_hbm, mask_hbm, xs_hbm, out_hbm,
          mask_v, xs_v, idx_v, win_v, acc_v, sem):
        c = lax.axis_index("c")
        s = lax.axis_index("s")
        b = s

        pltpu.sync_copy(mask_hbm.at[pl.ds(b * L, L)], mask_v)
        pltpu.sync_copy(xs_hbm, xs_v)

        hi_vec = jnp.zeros((_LANES,), jnp.int32)
        for i in range(L // _LANES):
            nz = mask_v[pl.ds(i * _LANES, _LANES)] != 0
            hi_vec = hi_vec + plsc.all_reduce_population_count(nz)

        x_vec = xs_v[...]
        lo_vec = jnp.maximum(hi_vec - x_vec, 0)

        lane = lax.iota(jnp.int32, _LANES)
        for g in range(_PAD // _LANES):
            r = jnp.minimum(lo_vec + (lane + g * _LANES), L - 1)
            idx_v[pl.ds(g * _LANES, _LANES)] = (b * L + r) * _NC + c
        pltpu.async_copy(hidden_hbm.at[idx_v], win_v, sem).wait()

        nwin_vec = hi_vec - lo_vec

        def row(j, acc):
            keep = (lane * 0 + j) < nwin_vec
            w = keep.astype(jnp.float32)
            return tuple(
                acc[t] + w * win_v[j, pl.ds(t * _LANES, _LANES)]
                for t in range(nchunk)
            )

        acc0 = tuple(jnp.zeros((_LANES,), jnp.float32) for _ in range(nchunk))
        acc = lax.fori_loop(0, _PAD, row, acc0)
        for t in range(nchunk):
            acc_v[pl.ds(t * _LANES, _LANES)] = acc[t]
        pltpu.sync_copy(acc_v, out_hbm.at[pl.ds(b * D + c * Dc, Dc)])

    return k(hidden2, mask_flat, xs)


def kernel(hidden, mask, x):
    B, L, D = hidden.shape
    assert B == _NS and D % (_NC * _LANES) == 0 and L % _LANES == 0
    xs = jnp.full((_LANES,), x, dtype=jnp.int32)
    mask_flat = mask.astype(jnp.int32).reshape(B * L)
    hidden2 = hidden.reshape(B * L * _NC, D // _NC)
    out = _sc_tail_sum(hidden2, mask_flat, xs, B, L, D)
    return out.reshape(B, 1, D).astype(hidden.dtype)


# speedup vs baseline: 6.5037x; 6.5037x over previous
import functools

import jax
import jax.numpy as jnp
from jax import lax
from jax.experimental import pallas as pl
from jax.experimental.pallas import tpu as pltpu
from jax.experimental.pallas import tpu_sc as plsc

_NC = 2
_NS = 16
_LANES = 16
_PAD = 64


def _sc_tail_sum(hidden, mask_flat, xs):
    B, L, D = hidden.shape
    Dc = D // _NC
    nchunk = Dc // _LANES
    mesh = plsc.VectorSubcoreMesh(core_axis_name="c", subcore_axis_name="s")

    @functools.partial(
        pl.kernel,
        out_type=jax.ShapeDtypeStruct((B * D,), jnp.float32),
        mesh=mesh,
        compiler_params=pltpu.CompilerParams(needs_layout_passes=False),
        scratch_types=[
            pltpu.VMEM((L,), jnp.int32),
            pltpu.VMEM((_LANES,), jnp.int32),
            pltpu.VMEM((_PAD, Dc), jnp.float32),
            pltpu.VMEM((Dc,), jnp.float32),
        ],
    )
    def k(hidden_hbm, mask_hbm, xs_hbm, out_hbm, mask_v, xs_v, win_v, acc_v):
        c = lax.axis_index("c")
        s = lax.axis_index("s")
        b = s

        pltpu.sync_copy(mask_hbm.at[pl.ds(b * L, L)], mask_v)
        pltpu.sync_copy(xs_hbm, xs_v)

        msum = jnp.zeros((_LANES,), jnp.int32)
        for i in range(L // _LANES):
            msum = msum + mask_v[pl.ds(i * _LANES, _LANES)]
        hi = jnp.sum(msum)
        x_s = jnp.max(xs_v[...])

        lo = jnp.maximum(hi - x_s, 0)
        base = jnp.minimum((lo // 8) * 8, L - _PAD)
        pltpu.sync_copy(
            hidden_hbm.at[b, pl.ds(base, _PAD), pl.ds(c * Dc, Dc)], win_v
        )

        lo_idx = lo - base
        hi_idx = hi - base

        def row(j, acc):
            return tuple(
                acc[t] + win_v[j, pl.ds(t * _LANES, _LANES)]
                for t in range(nchunk)
            )

        acc0 = tuple(jnp.zeros((_LANES,), jnp.float32) for _ in range(nchunk))
        acc = lax.fori_loop(lo_idx, hi_idx, row, acc0)
        for t in range(nchunk):
            acc_v[pl.ds(t * _LANES, _LANES)] = acc[t]
        pltpu.sync_copy(acc_v, out_hbm.at[pl.ds(b * D + c * Dc, Dc)])

    return k(hidden, mask_flat, xs)


def kernel(hidden, mask, x):
    B, L, D = hidden.shape
    assert B == _NS and D % (_NC * _LANES) == 0 and L % _LANES == 0
    assert L >= _PAD and L % 8 == 0
    xs = jnp.full((_LANES,), x, dtype=jnp.int32)
    mask_flat = mask.astype(jnp.int32).reshape(B * L)
    out = _sc_tail_sum(hidden, mask_flat, xs)
    return out.reshape(B, 1, D).astype(hidden.dtype)
